# baseline (speedup 1.0000x reference)
import functools

import jax
import jax.numpy as jnp
from jax import lax
from jax.experimental import pallas as pl
from jax.experimental.pallas import tpu as pltpu
from jax.experimental.pallas import tpu_sc as plsc

NUM_CLASSES = 1000000
EMBED_DIM = 64
BATCH = 16384

_NUM_CORES = 2
_NUM_SUBCORES = 16
_NUM_TILES = _NUM_CORES * _NUM_SUBCORES
_B_PER_TILE = BATCH // _NUM_TILES

_TR_W = 32768
_HALF = _TR_W // 2
_N_BLOCKS = (NUM_CLASSES + _TR_W - 1) // _TR_W
_P_ROWS = _N_BLOCKS * _HALF
_LOG2W = _TR_W.bit_length() - 1
_LOG2H = _LOG2W - 1
_QUARTER = _TR_W // 4


def _tr_pack_kernel(in_ref, out_ref):
    xb = jax.lax.bitcast_convert_type(
        in_ref[...].astype(jnp.bfloat16), jnp.uint16
    ).astype(jnp.uint32)
    lo = xb[:, :_HALF]
    hi = xb[:, _HALF:]
    packed = jax.lax.bitcast_convert_type(lo | (hi << 16), jnp.float32)
    out_ref[...] = jnp.concatenate(
        [packed[:, :_QUARTER].T, packed[:, _QUARTER:].T], axis=1
    )


def _tc_transpose_pack(tt, b0, nb):
    return pl.pallas_call(
        _tr_pack_kernel,
        grid=(nb,),
        in_specs=[pl.BlockSpec((EMBED_DIM, _TR_W), lambda i: (0, b0 + i))],
        out_specs=pl.BlockSpec((_QUARTER, 2 * EMBED_DIM), lambda i: (i, 0)),
        out_shape=jax.ShapeDtypeStruct(
            (nb * _QUARTER, 2 * EMBED_DIM), jnp.float32
        ),
        compiler_params=pltpu.CompilerParams(
            dimension_semantics=("arbitrary",)
        ),
    )(tt)


def _sc_gather(ptable, idx, row0, nrows):
    mesh = plsc.VectorSubcoreMesh(core_axis_name="c", subcore_axis_name="s")

    @functools.partial(
        pl.kernel,
        mesh=mesh,
        out_type=jax.ShapeDtypeStruct((BATCH, 2 * EMBED_DIM), ptable.dtype),
        scratch_types=[
            pltpu.VMEM((_B_PER_TILE,), jnp.int32),
            pltpu.VMEM((_B_PER_TILE, 2 * EMBED_DIM), jnp.float32),
            pltpu.SemaphoreType.DMA,
        ],
    )
    def gather_kernel(table_hbm, idx_hbm, out_hbm, idx_v, rows_v, sem):
        wid = lax.axis_index("s") * _NUM_CORES + lax.axis_index("c")
        base = wid * _B_PER_TILE
        pltpu.sync_copy(idx_hbm.at[pl.ds(base, _B_PER_TILE)], idx_v)

        @pl.loop(0, _B_PER_TILE, step=16)
        def _fire(j0):
            vi = idx_v[pl.ds(j0, 16)]
            vg = ((vi >> _LOG2W) << (_LOG2H - 1)) | (vi & (_QUARTER - 1))
            v = jnp.minimum(jnp.maximum(vg - row0, 0), nrows - 1)
            for k in range(16):
                pltpu.async_copy(
                    table_hbm.at[v[k]], rows_v.at[j0 + k], sem
                )

        @pl.loop(0, _B_PER_TILE)
        def _drain(j):
            pltpu.make_async_copy(
                table_hbm.at[0], rows_v.at[j], sem
            ).wait()

        pltpu.sync_copy(rows_v, out_hbm.at[pl.ds(base, _B_PER_TILE)])

    return gather_kernel(ptable, idx)


def _mlp_block_kernel(ea_ref, eb_ref, p_ref, w1_ref, b1_ref, w2_ref, b2_ref,
                      o_ref):
    sel = p_ref[...]
    ew = jnp.where((sel & 4) == 4, eb_ref[...], ea_ref[...])
    ewh = jnp.where((sel & 1) == 1, ew[:, EMBED_DIM:], ew[:, :EMBED_DIM])
    eu = jax.lax.bitcast_convert_type(ewh, jnp.uint32)
    even = jax.lax.bitcast_convert_type(eu << 16, jnp.float32)
    odd = jax.lax.bitcast_convert_type(eu & jnp.uint32(0xFFFF0000), jnp.float32)
    e = jnp.where((sel & 2) == 2, odd, even)
    h = jnp.dot(e, w1_ref[...], preferred_element_type=jnp.float32)
    h = h + b1_ref[...]
    h = h * jax.nn.sigmoid(h)
    h = jnp.dot(h, w2_ref[...], preferred_element_type=jnp.float32)
    h = h + b2_ref[...]
    o_ref[...] = (h * jax.nn.sigmoid(h)).T


def _tc_mlp(emb_a, emb_b, parity, W1t, b1, W2t, b2):
    blk = 2048
    grid = (BATCH // blk,)
    return pl.pallas_call(
        _mlp_block_kernel,
        grid=grid,
        in_specs=[
            pl.BlockSpec((blk, 2 * EMBED_DIM), lambda i: (i, 0)),
            pl.BlockSpec((blk, 2 * EMBED_DIM), lambda i: (i, 0)),
            pl.BlockSpec((blk, 1), lambda i: (i, 0)),
            pl.BlockSpec((EMBED_DIM, EMBED_DIM), lambda i: (0, 0)),
            pl.BlockSpec((1, EMBED_DIM), lambda i: (0, 0)),
            pl.BlockSpec((EMBED_DIM, EMBED_DIM), lambda i: (0, 0)),
            pl.BlockSpec((1, EMBED_DIM), lambda i: (0, 0)),
        ],
        out_specs=pl.BlockSpec((EMBED_DIM, blk), lambda i: (0, i)),
        out_shape=jax.ShapeDtypeStruct((EMBED_DIM, BATCH), jnp.float32),
    )(
        emb_a,
        emb_b,
        parity,
        W1t,
        b1.reshape(1, EMBED_DIM),
        W2t,
        b2.reshape(1, EMBED_DIM),
    )


_NB_A = _N_BLOCKS // 2
_NB_B = _N_BLOCKS - _NB_A
_SPLIT_CLASS = _NB_A * _TR_W


def kernel(x, table, W1, b1, W2, b2):
    idx = x.astype(jnp.int32)
    tt = table.T
    pa = _tc_transpose_pack(tt, 0, _NB_A)
    emb_a = _sc_gather(pa, idx, 0, _NB_A * _QUARTER)
    pb = _tc_transpose_pack(tt, _NB_A, _NB_B)
    emb_b = _sc_gather(pb, idx, _NB_A * _QUARTER, _NB_B * _QUARTER)
    sel = ((idx >> (_LOG2H - 1)) & 3) | jnp.where(
        idx >= _SPLIT_CLASS, 4, 0
    )
    out_cm = _tc_mlp(emb_a, emb_b, sel.reshape(BATCH, 1), W1.T, b1, W2.T, b2)
    return out_cm.T


# speedup vs baseline: 4.7122x; 4.7122x over previous
import functools

import jax
import jax.numpy as jnp
from jax import lax
from jax.experimental import pallas as pl
from jax.experimental.pallas import tpu as pltpu
from jax.experimental.pallas import tpu_sc as plsc

NUM_CLASSES = 1000000
EMBED_DIM = 64
BATCH = 16384

_NUM_CORES = 2
_NUM_SUBCORES = 16
_NUM_TILES = _NUM_CORES * _NUM_SUBCORES
_B_PER_TILE = BATCH // _NUM_TILES

_TR_W = 32768
_HALF = _TR_W // 2
_N_BLOCKS = (NUM_CLASSES + _TR_W - 1) // _TR_W
_P_ROWS = _N_BLOCKS * _HALF
_LOG2W = _TR_W.bit_length() - 1
_LOG2H = _LOG2W - 1
_QUARTER = _TR_W // 4


def _tr_pack_kernel(in_ref, out_ref):
    xb = jax.lax.bitcast_convert_type(
        in_ref[...].astype(jnp.bfloat16), jnp.uint16
    ).astype(jnp.uint32)
    lo = xb[:, :_HALF]
    hi = xb[:, _HALF:]
    packed = jax.lax.bitcast_convert_type(lo | (hi << 16), jnp.float32)
    out_ref[...] = jnp.concatenate(
        [packed[:, :_QUARTER].T, packed[:, _QUARTER:].T], axis=1
    )


def _tc_transpose_pack(table):
    tt = table.T
    return pl.pallas_call(
        _tr_pack_kernel,
        grid=(_N_BLOCKS,),
        in_specs=[pl.BlockSpec((EMBED_DIM, _TR_W), lambda i: (0, i))],
        out_specs=pl.BlockSpec((_QUARTER, 2 * EMBED_DIM), lambda i: (i, 0)),
        out_shape=jax.ShapeDtypeStruct(
            (_N_BLOCKS * _QUARTER, 2 * EMBED_DIM), jnp.float32
        ),
        compiler_params=pltpu.CompilerParams(
            dimension_semantics=("arbitrary",)
        ),
    )(tt)


def _sc_gather(ptable, idx):
    mesh = plsc.VectorSubcoreMesh(core_axis_name="c", subcore_axis_name="s")

    @functools.partial(
        pl.kernel,
        mesh=mesh,
        out_type=jax.ShapeDtypeStruct((BATCH, 2 * EMBED_DIM), ptable.dtype),
        scratch_types=[
            pltpu.VMEM((_B_PER_TILE,), jnp.int32),
            pltpu.VMEM((_B_PER_TILE, 2 * EMBED_DIM), jnp.float32),
            pltpu.SemaphoreType.DMA,
        ],
    )
    def gather_kernel(table_hbm, idx_hbm, out_hbm, idx_v, rows_v, sem):
        wid = lax.axis_index("s") * _NUM_CORES + lax.axis_index("c")
        base = wid * _B_PER_TILE
        pltpu.sync_copy(idx_hbm.at[pl.ds(base, _B_PER_TILE)], idx_v)

        @pl.loop(0, _B_PER_TILE, step=16)
        def _fire(j0):
            vi = idx_v[pl.ds(j0, 16)]
            v = ((vi >> _LOG2W) << (_LOG2H - 1)) | (vi & (_QUARTER - 1))
            for k in range(16):
                pltpu.async_copy(
                    table_hbm.at[v[k]], rows_v.at[j0 + k], sem
                )

        @pl.loop(0, _B_PER_TILE)
        def _drain(j):
            pltpu.make_async_copy(
                table_hbm.at[0], rows_v.at[j], sem
            ).wait()

        pltpu.sync_copy(rows_v, out_hbm.at[pl.ds(base, _B_PER_TILE)])

    return gather_kernel(ptable, idx)


def _mlp_block_kernel(e_ref, p_ref, w1_ref, b1_ref, w2_ref, b2_ref, o_ref):
    ew = e_ref[...]
    sel = p_ref[...]
    ewh = jnp.where((sel & 1) == 1, ew[:, EMBED_DIM:], ew[:, :EMBED_DIM])
    eu = jax.lax.bitcast_convert_type(ewh, jnp.uint32)
    even = jax.lax.bitcast_convert_type(eu << 16, jnp.float32)
    odd = jax.lax.bitcast_convert_type(eu & jnp.uint32(0xFFFF0000), jnp.float32)
    e = jnp.where((sel & 2) == 2, odd, even)
    h = jnp.dot(e, w1_ref[...], preferred_element_type=jnp.float32)
    h = h + b1_ref[...]
    h = h * jax.nn.sigmoid(h)
    h = jnp.dot(h, w2_ref[...], preferred_element_type=jnp.float32)
    h = h + b2_ref[...]
    o_ref[...] = (h * jax.nn.sigmoid(h)).T


def _tc_mlp(emb, parity, W1t, b1, W2t, b2):
    blk = 2048
    grid = (BATCH // blk,)
    return pl.pallas_call(
        _mlp_block_kernel,
        grid=grid,
        in_specs=[
            pl.BlockSpec((blk, 2 * EMBED_DIM), lambda i: (i, 0)),
            pl.BlockSpec((blk, 1), lambda i: (i, 0)),
            pl.BlockSpec((EMBED_DIM, EMBED_DIM), lambda i: (0, 0)),
            pl.BlockSpec((1, EMBED_DIM), lambda i: (0, 0)),
            pl.BlockSpec((EMBED_DIM, EMBED_DIM), lambda i: (0, 0)),
            pl.BlockSpec((1, EMBED_DIM), lambda i: (0, 0)),
        ],
        out_specs=pl.BlockSpec((EMBED_DIM, blk), lambda i: (0, i)),
        out_shape=jax.ShapeDtypeStruct((EMBED_DIM, BATCH), jnp.float32),
    )(
        emb,
        parity,
        W1t,
        b1.reshape(1, EMBED_DIM),
        W2t,
        b2.reshape(1, EMBED_DIM),
    )


def kernel(x, table, W1, b1, W2, b2):
    idx = x.astype(jnp.int32)
    ptable = _tc_transpose_pack(table)
    emb = _sc_gather(ptable, idx)
    parity = ((idx >> (_LOG2H - 1)) & 3).reshape(BATCH, 1)
    out_cm = _tc_mlp(emb, parity, W1.T, b1, W2.T, b2)
    return out_cm.T
